# baseline (speedup 1.0000x reference)
import functools

import jax
import jax.numpy as jnp
import numpy as np
from jax import lax
from jax.experimental import pallas as pl
from jax.experimental.pallas import tpu as pltpu

HID = 128
HEADS = 4
HD = 32
ED = 16
N = 25000
E = 100000
E_PAD = 106496
CH = 128
G_PER_W = 3328
G_STEPS = G_PER_W // CH
S_PER_T = E_PAD // 16
S_CH = 128
S_STEPS = S_PER_T // S_CH
NHALF = 12544
DUMP = NHALF
ACC = NHALF + 8
N_PAD = 2 * NHALF
TPS = NHALF // 16
SENTINEL = 1 << 28

TYPE_TRIPLES = [
    ("ll_temporal", "left", "left"),
    ("rr_temporal", "right", "right"),
    ("ll_context", "left", "left"),
    ("rr_context", "right", "right"),
    ("lr_bimanual", "left", "right"),
    ("rl_bimanual", "right", "left"),
]


def _np_hsel():
    m = np.zeros((HID, 16), np.float32)
    for d in range(HID):
        m[d, d // HD] = 1.0 / np.sqrt(HD)
    return m


def _np_expand():
    m = np.zeros((16, HID), np.float32)
    for d in range(HID):
        m[d // HD, d] = 1.0
    return m


def _proj_body(x, w, q0, q1, q2, kv0, kv1, kv2):
    y = jnp.dot(x[...], w[...], preferred_element_type=jnp.float32)
    q0[...] = y[:, 0:128]
    q1[...] = y[:, 128:256]
    q2[...] = y[:, 256:384]
    kv0[...] = y[:, 384:640]
    kv1[...] = y[:, 640:896]
    kv2[...] = y[:, 896:1152]


def _proj(x, wcat):
    r = 200
    grid = (N // r,)
    return pl.pallas_call(
        _proj_body,
        grid=grid,
        in_specs=[
            pl.BlockSpec((r, HID), lambda i: (i, 0)),
            pl.BlockSpec((HID, 1152), lambda i: (0, 0)),
        ],
        out_specs=[pl.BlockSpec((r, HID), lambda i: (i, 0))] * 3
        + [pl.BlockSpec((r, 256), lambda i: (i, 0))] * 3,
        out_shape=[jax.ShapeDtypeStruct((N, HID), jnp.float32)] * 3
        + [jax.ShapeDtypeStruct((N, 256), jnp.float32)] * 3,
    )(x, wcat)


def _edge_body(qd, kvs, ea, wekv, hsel, expand, numr, exr):
    kv = kvs[...] + jnp.dot(ea[...], wekv[...], preferred_element_type=jnp.float32)
    k = kv[:, :HID]
    v = kv[:, HID:]
    s16 = jnp.dot(qd[...] * k, hsel[...], preferred_element_type=jnp.float32)
    ex16 = jnp.exp(s16)
    ex128 = jnp.dot(ex16, expand[...], preferred_element_type=jnp.float32)
    exr[...] = ex128
    numr[...] = ex128 * v


def _edge(qd, kvs, ea, wekv, hsel, expand):
    r = 512
    grid = (E_PAD // r,)
    return pl.pallas_call(
        _edge_body,
        grid=grid,
        in_specs=[
            pl.BlockSpec((r, HID), lambda i: (i, 0)),
            pl.BlockSpec((r, 256), lambda i: (i, 0)),
            pl.BlockSpec((r, ED), lambda i: (i, 0)),
            pl.BlockSpec((ED, 256), lambda i: (0, 0)),
            pl.BlockSpec((HID, 16), lambda i: (0, 0)),
            pl.BlockSpec((16, HID), lambda i: (0, 0)),
        ],
        out_specs=[
            pl.BlockSpec((r, HID), lambda i: (i, 0)),
            pl.BlockSpec((r, HID), lambda i: (i, 0)),
        ],
        out_shape=[
            jax.ShapeDtypeStruct((E_PAD, HID), jnp.float32),
            jax.ShapeDtypeStruct((E_PAD, HID), jnp.float32),
        ],
    )(qd, kvs, ea, wekv, hsel, expand)


def _node_body(x, n0, n1, n2, d0, d1, d2, wo, ln1s, ln1b, w1, b1, w2,
               b2, ln2s, ln2b, out):
    agg = (n0[...] / (d0[...] + 1e-9)
           + n1[...] / (d1[...] + 1e-9)
           + n2[...] / (d2[...] + 1e-9))
    h1 = x[...] + jnp.dot(agg, wo[...], preferred_element_type=jnp.float32)
    mu = jnp.mean(h1, axis=1, keepdims=True)
    var = jnp.mean((h1 - mu) ** 2, axis=1, keepdims=True)
    h = (h1 - mu) * lax.rsqrt(var + 1e-5) * ln1s[...] + ln1b[...]
    f = jnp.maximum(jnp.dot(h, w1[...], preferred_element_type=jnp.float32) + b1[...], 0.0)
    f = jnp.dot(f, w2[...], preferred_element_type=jnp.float32) + b2[...]
    h2 = h + f
    mu2 = jnp.mean(h2, axis=1, keepdims=True)
    var2 = jnp.mean((h2 - mu2) ** 2, axis=1, keepdims=True)
    out[...] = (h2 - mu2) * lax.rsqrt(var2 + 1e-5) * ln2s[...] + ln2b[...]


def _node(x, n0, n1, n2, d0, d1, d2, npar):
    r = 200
    grid = (N // r,)
    row = lambda i: (i, 0)
    fix = lambda i: (0, 0)
    return pl.pallas_call(
        _node_body,
        grid=grid,
        in_specs=[
            pl.BlockSpec((r, HID), row),
            pl.BlockSpec((r, HID), row),
            pl.BlockSpec((r, HID), row),
            pl.BlockSpec((r, HID), row),
            pl.BlockSpec((r, HID), row),
            pl.BlockSpec((r, HID), row),
            pl.BlockSpec((r, HID), row),
            pl.BlockSpec((HID, HID), fix),
            pl.BlockSpec((1, HID), fix),
            pl.BlockSpec((1, HID), fix),
            pl.BlockSpec((HID, 4 * HID), fix),
            pl.BlockSpec((1, 4 * HID), fix),
            pl.BlockSpec((4 * HID, HID), fix),
            pl.BlockSpec((1, HID), fix),
            pl.BlockSpec((1, HID), fix),
            pl.BlockSpec((1, HID), fix),
        ],
        out_specs=pl.BlockSpec((r, HID), row),
        out_shape=jax.ShapeDtypeStruct((N, HID), jnp.float32),
    )(
        x, n0, n1, n2, d0, d1, d2, npar["Wo"],
        npar["ln1_s"].reshape(1, HID), npar["ln1_b"].reshape(1, HID),
        npar["W1"], npar["b1"].reshape(1, 4 * HID),
        npar["W2"], npar["b2"].reshape(1, HID),
        npar["ln2_s"].reshape(1, HID), npar["ln2_b"].reshape(1, HID),
    )


@functools.lru_cache(maxsize=None)
def _sc_gather_kernel():
    from jax.experimental.pallas import tpu_sc as plsc

    mesh = plsc.VectorSubcoreMesh(core_axis_name="c", subcore_axis_name="s")

    @functools.partial(
        pl.kernel,
        out_type=(
            jax.ShapeDtypeStruct((E_PAD, HID), jnp.float32),
            jax.ShapeDtypeStruct((E_PAD, 256), jnp.float32),
        ),
        mesh=mesh,
        scratch_types=[
            pltpu.VMEM((G_STEPS, CH), jnp.int32),
            pltpu.VMEM((G_STEPS, CH), jnp.int32),
            pltpu.VMEM((2, CH, HID), jnp.float32),
            pltpu.VMEM((2, CH, 256), jnp.float32),
            pltpu.SemaphoreType.DMA,
            pltpu.SemaphoreType.DMA,
            pltpu.SemaphoreType.DMA,
            pltpu.SemaphoreType.DMA,
        ],
    )
    def gather(qtab, kvtab, dst2, src2, qd_out, kvs_out, idxd, idxs, qbuf,
               kvbuf, sq0, sq1, sk0, sk1):
        wid = lax.axis_index("s") * 2 + lax.axis_index("c")
        base = wid * G_PER_W
        sq = (sq0, sq1)
        sk = (sk0, sk1)
        pltpu.sync_copy(dst2.at[wid], idxd)
        pltpu.sync_copy(src2.at[wid], idxs)
        for p in (0, 1):
            pltpu.async_copy(qtab.at[idxd.at[p]], qbuf.at[p], sq[p])
            pltpu.async_copy(kvtab.at[idxs.at[p]], kvbuf.at[p], sk[p])

        def body(ci2, carry):
            for p in (0, 1):
                ci = ci2 * 2 + p
                off = base + ci * CH
                pltpu.make_async_copy(qtab.at[idxd.at[ci]], qbuf.at[p],
                                      sq[p]).wait()
                pltpu.make_async_copy(kvtab.at[idxs.at[ci]], kvbuf.at[p],
                                      sk[p]).wait()
                pltpu.sync_copy(qbuf.at[p], qd_out.at[pl.ds(off, CH)])
                pltpu.sync_copy(kvbuf.at[p], kvs_out.at[pl.ds(off, CH)])

                @pl.when(ci + 2 < G_STEPS)
                def _next():
                    pltpu.async_copy(qtab.at[idxd.at[ci + 2]], qbuf.at[p],
                                     sq[p])
                    pltpu.async_copy(kvtab.at[idxs.at[ci + 2]], kvbuf.at[p],
                                     sk[p])

            return carry

        lax.fori_loop(0, G_STEPS // 2, body, 0)

    return gather


@functools.lru_cache(maxsize=None)
def _sc_scatter_kernel(width):
    from jax.experimental.pallas import tpu_sc as plsc

    mesh = plsc.VectorSubcoreMesh(core_axis_name="c", subcore_axis_name="s")

    @functools.partial(
        pl.kernel,
        out_type=jax.ShapeDtypeStruct((N_PAD, width), jnp.float32),
        mesh=mesh,
        scratch_types=[
            pltpu.VMEM((S_STEPS, S_CH), jnp.int32),
            pltpu.VMEM((S_CH, width), jnp.float32),
            pltpu.VMEM_SHARED((ACC, width), jnp.float32),
        ],
    )
    def scatter(rows, idx4, zrow, out, idxl, rbuf, accn):
        c = lax.axis_index("c")
        s = lax.axis_index("s")
        nbase = c * NHALF
        tbase = s * S_PER_T
        pltpu.sync_copy(idx4.at[c, s], idxl)
        pltpu.sync_copy(zrow.at[pl.ds(0, TPS)], accn.at[pl.ds(s * TPS, TPS)])

        @pl.when(s == 0)
        def _zero_dump():
            pltpu.sync_copy(zrow.at[pl.ds(TPS, 8)], accn.at[pl.ds(NHALF, 8)])

        plsc.subcore_barrier()

        def body(ci, carry):
            off = tbase + ci * S_CH
            pltpu.sync_copy(rows.at[pl.ds(off, S_CH)], rbuf)
            pltpu.sync_copy(rbuf, accn.at[idxl.at[ci]], add=True)
            return carry

        lax.fori_loop(0, S_STEPS, body, 0)
        plsc.subcore_barrier()
        pltpu.sync_copy(accn.at[pl.ds(s * TPS, TPS)],
                        out.at[pl.ds(nbase + s * TPS, TPS)])

    return scatter


def kernel(all_gripper_feats_left, all_gripper_feats_right, edge_index,
           edge_attr, params, current_start_left, current_end_left,
           current_start_right, current_end_right):
    hsel = jnp.asarray(_np_hsel())
    expand = jnp.asarray(_np_expand())
    zn = jnp.zeros((TPS + 8, HID), jnp.float32)

    pad = E_PAD - E
    srcs, dst_g, dst_s, ea_p = {}, {}, {}, {}
    for name, _, _ in TYPE_TRIPLES:
        srcs[name] = jnp.pad(edge_index[name][0],
                             (0, pad)).reshape(32, G_STEPS, CH)
        dst_g[name] = jnp.pad(edge_index[name][1],
                              (0, pad)).reshape(32, G_STEPS, CH)
        dstp = jnp.pad(edge_index[name][1], (0, pad),
                       constant_values=SENTINEL)
        halves = []
        for cidx in (0, 1):
            lo = cidx * NHALF
            loc = dstp - lo
            ok = (dstp >= lo) & (dstp < lo + NHALF)
            halves.append(jnp.where(ok, loc, DUMP))
        dst_s[name] = jnp.stack(halves).astype(jnp.int32).reshape(
            2, 16, S_STEPS, S_CH)
        ea_p[name] = jnp.pad(edge_attr[name], ((0, pad), (0, 0)))

    gather = _sc_gather_kernel()
    scatter_add = _sc_scatter_kernel(HID)

    x = {"left": all_gripper_feats_left, "right": all_gripper_feats_right}
    for lp in params["layers"]:
        tabs = {}
        for side in ("left", "right"):
            dst_ts = [t for t, _, d in TYPE_TRIPLES if d == side]
            src_ts = [t for t, s, _ in TYPE_TRIPLES if s == side]
            wcat = jnp.concatenate(
                [lp[t]["Wq"] for t in dst_ts]
                + [w for t in src_ts for w in (lp[t]["Wk"], lp[t]["Wv"])],
                axis=1)
            outs = _proj(x[side], wcat)
            for t, qt in zip(dst_ts, outs[:3]):
                tabs[("q", t)] = qt
            for t, kvt in zip(src_ts, outs[3:]):
                tabs[("kv", t)] = kvt

        aggs = {"left": [], "right": []}
        for name, st, dt in TYPE_TRIPLES:
            qd, kvs = gather(tabs[("q", name)], tabs[("kv", name)],
                             dst_g[name], srcs[name])
            wekv = jnp.concatenate([lp[name]["We_k"], lp[name]["We_v"]],
                                   axis=1)
            numr, exr = _edge(qd, kvs, ea_p[name], wekv, hsel, expand)
            num = scatter_add(numr, dst_s[name], zn)
            den = scatter_add(exr, dst_s[name], zn)
            aggs[dt].append((num[:N], den[:N]))

        newx = {}
        for side in ("left", "right"):
            (n0, d0), (n1, d1), (n2, d2) = aggs[side]
            newx[side] = _node(x[side], n0, n1, n2, d0, d1, d2,
                               lp["node_" + side])
        x = newx

    bl = lax.dynamic_slice_in_dim(x["left"], current_end_left - 1000, 1000, axis=0)
    br = lax.dynamic_slice_in_dim(x["right"], current_end_right - 1000, 1000, axis=0)
    return (bl, br)


# speedup vs baseline: 1.1663x; 1.1663x over previous
import functools

import jax
import jax.numpy as jnp
import numpy as np
from jax import lax
from jax.experimental import pallas as pl
from jax.experimental.pallas import tpu as pltpu

HID = 128
HEADS = 4
HD = 32
ED = 16
N = 25000
E = 100000
E_PAD = 102400
CH = 128
G_PER_W = 3200
G_STEPS = G_PER_W // CH
S_PER_T = E_PAD // 16
S_STEPS = S_PER_T // CH
NHALF = 12800
DUMP = NHALF
ACC = NHALF + 8
N_PAD = 2 * NHALF
TPS = NHALF // 16
SENTINEL = 1 << 28

TYPE_TRIPLES = [
    ("ll_temporal", "left", "left"),
    ("rr_temporal", "right", "right"),
    ("ll_context", "left", "left"),
    ("rr_context", "right", "right"),
    ("lr_bimanual", "left", "right"),
    ("rl_bimanual", "right", "left"),
]


def _np_hsel():
    m = np.zeros((HID, 16), np.float32)
    for d in range(HID):
        m[d, d // HD] = 1.0 / np.sqrt(HD)
    return m


def _np_expand():
    m = np.zeros((16, HID), np.float32)
    for d in range(HID):
        m[d // HD, d] = 1.0
    return m


def _proj_body(x, w, q0, q1, q2, kv0, kv1, kv2):
    y = jnp.dot(x[...], w[...], preferred_element_type=jnp.float32)
    q0[...] = y[:, 0:128]
    q1[...] = y[:, 128:256]
    q2[...] = y[:, 256:384]
    kv0[...] = y[:, 384:640]
    kv1[...] = y[:, 640:896]
    kv2[...] = y[:, 896:1152]


def _proj(x, wcat):
    r = 200
    grid = (N // r,)
    return pl.pallas_call(
        _proj_body,
        grid=grid,
        in_specs=[
            pl.BlockSpec((r, HID), lambda i: (i, 0)),
            pl.BlockSpec((HID, 1152), lambda i: (0, 0)),
        ],
        out_specs=[pl.BlockSpec((r, HID), lambda i: (i, 0))] * 3
        + [pl.BlockSpec((r, 256), lambda i: (i, 0))] * 3,
        out_shape=[jax.ShapeDtypeStruct((N, HID), jnp.float32)] * 3
        + [jax.ShapeDtypeStruct((N, 256), jnp.float32)] * 3,
    )(x, wcat)


def _edge_body(qd, kvs, ea, wekv, hsel, expand, numr, exr):
    kv = kvs[...] + jnp.dot(ea[...], wekv[...], preferred_element_type=jnp.float32)
    k = kv[:, :HID]
    v = kv[:, HID:]
    s16 = jnp.dot(qd[...] * k, hsel[...], preferred_element_type=jnp.float32)
    ex16 = jnp.exp(s16)
    ex128 = jnp.dot(ex16, expand[...], preferred_element_type=jnp.float32)
    exr[...] = ex128
    numr[...] = ex128 * v


def _edge(qd, kvs, ea, wekv, hsel, expand):
    r = 512
    grid = (E_PAD // r,)
    return pl.pallas_call(
        _edge_body,
        grid=grid,
        in_specs=[
            pl.BlockSpec((r, HID), lambda i: (i, 0)),
            pl.BlockSpec((r, 256), lambda i: (i, 0)),
            pl.BlockSpec((r, ED), lambda i: (i, 0)),
            pl.BlockSpec((ED, 256), lambda i: (0, 0)),
            pl.BlockSpec((HID, 16), lambda i: (0, 0)),
            pl.BlockSpec((16, HID), lambda i: (0, 0)),
        ],
        out_specs=[
            pl.BlockSpec((r, HID), lambda i: (i, 0)),
            pl.BlockSpec((r, HID), lambda i: (i, 0)),
        ],
        out_shape=[
            jax.ShapeDtypeStruct((E_PAD, HID), jnp.float32),
            jax.ShapeDtypeStruct((E_PAD, HID), jnp.float32),
        ],
    )(qd, kvs, ea, wekv, hsel, expand)


def _node_body(x, n0, n1, n2, d0, d1, d2, wo, ln1s, ln1b, w1, b1, w2,
               b2, ln2s, ln2b, out):
    agg = (n0[...] / (d0[...] + 1e-9)
           + n1[...] / (d1[...] + 1e-9)
           + n2[...] / (d2[...] + 1e-9))
    h1 = x[...] + jnp.dot(agg, wo[...], preferred_element_type=jnp.float32)
    mu = jnp.mean(h1, axis=1, keepdims=True)
    var = jnp.mean((h1 - mu) ** 2, axis=1, keepdims=True)
    h = (h1 - mu) * lax.rsqrt(var + 1e-5) * ln1s[...] + ln1b[...]
    f = jnp.maximum(jnp.dot(h, w1[...], preferred_element_type=jnp.float32) + b1[...], 0.0)
    f = jnp.dot(f, w2[...], preferred_element_type=jnp.float32) + b2[...]
    h2 = h + f
    mu2 = jnp.mean(h2, axis=1, keepdims=True)
    var2 = jnp.mean((h2 - mu2) ** 2, axis=1, keepdims=True)
    out[...] = (h2 - mu2) * lax.rsqrt(var2 + 1e-5) * ln2s[...] + ln2b[...]


def _node(x, n0, n1, n2, d0, d1, d2, npar):
    r = 200
    grid = (N // r,)
    row = lambda i: (i, 0)
    fix = lambda i: (0, 0)
    return pl.pallas_call(
        _node_body,
        grid=grid,
        in_specs=[
            pl.BlockSpec((r, HID), row),
            pl.BlockSpec((r, HID), row),
            pl.BlockSpec((r, HID), row),
            pl.BlockSpec((r, HID), row),
            pl.BlockSpec((r, HID), row),
            pl.BlockSpec((r, HID), row),
            pl.BlockSpec((r, HID), row),
            pl.BlockSpec((HID, HID), fix),
            pl.BlockSpec((1, HID), fix),
            pl.BlockSpec((1, HID), fix),
            pl.BlockSpec((HID, 4 * HID), fix),
            pl.BlockSpec((1, 4 * HID), fix),
            pl.BlockSpec((4 * HID, HID), fix),
            pl.BlockSpec((1, HID), fix),
            pl.BlockSpec((1, HID), fix),
            pl.BlockSpec((1, HID), fix),
        ],
        out_specs=pl.BlockSpec((r, HID), row),
        out_shape=jax.ShapeDtypeStruct((N, HID), jnp.float32),
    )(
        x, n0, n1, n2, d0, d1, d2, npar["Wo"],
        npar["ln1_s"].reshape(1, HID), npar["ln1_b"].reshape(1, HID),
        npar["W1"], npar["b1"].reshape(1, 4 * HID),
        npar["W2"], npar["b2"].reshape(1, HID),
        npar["ln2_s"].reshape(1, HID), npar["ln2_b"].reshape(1, HID),
    )


@functools.lru_cache(maxsize=None)
def _sc_gather_kernel():
    from jax.experimental.pallas import tpu_sc as plsc

    mesh = plsc.VectorSubcoreMesh(core_axis_name="c", subcore_axis_name="s")

    @functools.partial(
        pl.kernel,
        out_type=(
            jax.ShapeDtypeStruct((E_PAD, HID), jnp.float32),
            jax.ShapeDtypeStruct((E_PAD, 256), jnp.float32),
        ),
        mesh=mesh,
        scratch_types=[
            pltpu.VMEM((CH,), jnp.int32),
            pltpu.VMEM((CH,), jnp.int32),
            pltpu.VMEM((CH, HID), jnp.float32),
            pltpu.VMEM((CH, 256), jnp.float32),
            pltpu.SemaphoreType.DMA,
            pltpu.SemaphoreType.DMA,
        ],
    )
    def gather(qtab, kvtab, dstp, srcp, qd_out, kvs_out, idxd, idxs, qbuf,
               kvbuf, sq, skv):
        wid = lax.axis_index("s") * 2 + lax.axis_index("c")
        base = wid * G_PER_W

        def body(ci, carry):
            off = base + ci * CH
            pltpu.sync_copy(dstp.at[pl.ds(off, CH)], idxd)
            pltpu.sync_copy(srcp.at[pl.ds(off, CH)], idxs)
            cq = pltpu.async_copy(qtab.at[idxd], qbuf, sq)
            ck = pltpu.async_copy(kvtab.at[idxs], kvbuf, skv)
            cq.wait()
            ck.wait()
            pltpu.sync_copy(qbuf, qd_out.at[pl.ds(off, CH)])
            pltpu.sync_copy(kvbuf, kvs_out.at[pl.ds(off, CH)])
            return carry

        lax.fori_loop(0, G_STEPS, body, 0)

    return gather


@functools.lru_cache(maxsize=None)
def _sc_scatter_kernel(width):
    from jax.experimental.pallas import tpu_sc as plsc

    mesh = plsc.VectorSubcoreMesh(core_axis_name="c", subcore_axis_name="s")

    @functools.partial(
        pl.kernel,
        out_type=jax.ShapeDtypeStruct((N_PAD, width), jnp.float32),
        mesh=mesh,
        scratch_types=[
            pltpu.VMEM((1, CH), jnp.int32),
            pltpu.VMEM((CH, width), jnp.float32),
            pltpu.VMEM_SHARED((ACC, width), jnp.float32),
        ],
    )
    def scatter(rows, idx2, zrow, out, idxl, rbuf, accn):
        c = lax.axis_index("c")
        s = lax.axis_index("s")
        nbase = c * NHALF
        pltpu.sync_copy(zrow.at[pl.ds(0, TPS)], accn.at[pl.ds(s * TPS, TPS)])

        @pl.when(s == 0)
        def _zero_dump():
            pltpu.sync_copy(zrow.at[pl.ds(TPS, 8)], accn.at[pl.ds(NHALF, 8)])

        plsc.subcore_barrier()

        def body(ci, carry):
            off = s * S_PER_T + ci * CH
            pltpu.sync_copy(idx2.at[c, pl.ds(off, CH)], idxl.at[0])
            pltpu.sync_copy(rows.at[pl.ds(off, CH)], rbuf)
            pltpu.sync_copy(rbuf, accn.at[idxl.at[0]], add=True)
            return carry

        lax.fori_loop(0, S_STEPS, body, 0)
        plsc.subcore_barrier()
        pltpu.sync_copy(accn.at[pl.ds(s * TPS, TPS)],
                        out.at[pl.ds(nbase + s * TPS, TPS)])

    return scatter


def kernel(all_gripper_feats_left, all_gripper_feats_right, edge_index,
           edge_attr, params, current_start_left, current_end_left,
           current_start_right, current_end_right):
    hsel = jnp.asarray(_np_hsel())
    expand = jnp.asarray(_np_expand())
    zn = jnp.zeros((TPS + 8, HID), jnp.float32)

    pad = E_PAD - E
    srcs, dst_g, dst_s, ea_p = {}, {}, {}, {}
    for name, _, _ in TYPE_TRIPLES:
        srcs[name] = jnp.pad(edge_index[name][0], (0, pad))
        dst_g[name] = jnp.pad(edge_index[name][1], (0, pad))
        dstp = jnp.pad(edge_index[name][1], (0, pad),
                       constant_values=SENTINEL)
        halves = []
        for cidx in (0, 1):
            lo = cidx * NHALF
            loc = dstp - lo
            ok = (dstp >= lo) & (dstp < lo + NHALF)
            halves.append(jnp.where(ok, loc, DUMP))
        dst_s[name] = jnp.stack(halves).astype(jnp.int32)
        ea_p[name] = jnp.pad(edge_attr[name], ((0, pad), (0, 0)))

    gather = _sc_gather_kernel()
    scatter_add = _sc_scatter_kernel(HID)

    x = {"left": all_gripper_feats_left, "right": all_gripper_feats_right}
    for lp in params["layers"]:
        tabs = {}
        for side in ("left", "right"):
            dst_ts = [t for t, _, d in TYPE_TRIPLES if d == side]
            src_ts = [t for t, s, _ in TYPE_TRIPLES if s == side]
            wcat = jnp.concatenate(
                [lp[t]["Wq"] for t in dst_ts]
                + [w for t in src_ts for w in (lp[t]["Wk"], lp[t]["Wv"])],
                axis=1)
            outs = _proj(x[side], wcat)
            for t, qt in zip(dst_ts, outs[:3]):
                tabs[("q", t)] = qt
            for t, kvt in zip(src_ts, outs[3:]):
                tabs[("kv", t)] = kvt

        aggs = {"left": [], "right": []}
        for name, st, dt in TYPE_TRIPLES:
            qd, kvs = gather(tabs[("q", name)], tabs[("kv", name)],
                             dst_g[name], srcs[name])
            wekv = jnp.concatenate([lp[name]["We_k"], lp[name]["We_v"]],
                                   axis=1)
            numr, exr = _edge(qd, kvs, ea_p[name], wekv, hsel, expand)
            num = scatter_add(numr, dst_s[name], zn)
            den = scatter_add(exr, dst_s[name], zn)
            aggs[dt].append((num[:N], den[:N]))

        newx = {}
        for side in ("left", "right"):
            (n0, d0), (n1, d1), (n2, d2) = aggs[side]
            newx[side] = _node(x[side], n0, n1, n2, d0, d1, d2,
                               lp["node_" + side])
        x = newx

    bl = lax.dynamic_slice_in_dim(x["left"], current_end_left - 1000, 1000, axis=0)
    br = lax.dynamic_slice_in_dim(x["right"], current_end_right - 1000, 1000, axis=0)
    return (bl, br)


# speedup vs baseline: 1.2082x; 1.0360x over previous
import functools

import jax
import jax.numpy as jnp
import numpy as np
from jax import lax
from jax.experimental import pallas as pl
from jax.experimental.pallas import tpu as pltpu

HID = 128
HEADS = 4
HD = 32
ED = 16
N = 25000
E = 100000
E_PAD = 102400
CH = 128
G_PER_W = 3200
G_STEPS = G_PER_W // CH
S_PER_T = E_PAD // 16
S_STEPS = S_PER_T // CH
NHALF = 12800
DUMP = NHALF
ACC = NHALF + 8
N_PAD = 2 * NHALF
TPS = NHALF // 16
SENTINEL = 1 << 28

TYPE_TRIPLES = [
    ("ll_temporal", "left", "left"),
    ("rr_temporal", "right", "right"),
    ("ll_context", "left", "left"),
    ("rr_context", "right", "right"),
    ("lr_bimanual", "left", "right"),
    ("rl_bimanual", "right", "left"),
]


def _np_hsel():
    m = np.zeros((HID, 16), np.float32)
    for d in range(HID):
        m[d, d // HD] = 1.0 / np.sqrt(HD)
    return m


def _np_expand():
    m = np.zeros((16, HID), np.float32)
    for d in range(HID):
        m[d // HD, d] = 1.0
    return m


def _proj_body(x, w, q0, q1, q2, kv0, kv1, kv2):
    y = jnp.dot(x[...], w[...], preferred_element_type=jnp.float32)
    q0[...] = y[:, 0:128]
    q1[...] = y[:, 128:256]
    q2[...] = y[:, 256:384]
    kv0[...] = y[:, 384:640]
    kv1[...] = y[:, 640:896]
    kv2[...] = y[:, 896:1152]


def _proj(x, wcat):
    r = 200
    grid = (N // r,)
    return pl.pallas_call(
        _proj_body,
        grid=grid,
        in_specs=[
            pl.BlockSpec((r, HID), lambda i: (i, 0)),
            pl.BlockSpec((HID, 1152), lambda i: (0, 0)),
        ],
        out_specs=[pl.BlockSpec((r, HID), lambda i: (i, 0))] * 3
        + [pl.BlockSpec((r, 256), lambda i: (i, 0))] * 3,
        out_shape=[jax.ShapeDtypeStruct((N, HID), jnp.float32)] * 3
        + [jax.ShapeDtypeStruct((N, 256), jnp.float32)] * 3,
    )(x, wcat)


def _edge_body(qd, kvs, ea, wekv, hsel, expand, numr, exr):
    kv = kvs[...] + jnp.dot(ea[...], wekv[...], preferred_element_type=jnp.float32)
    k = kv[:, :HID]
    v = kv[:, HID:]
    s16 = jnp.dot(qd[...] * k, hsel[...], preferred_element_type=jnp.float32)
    ex16 = jnp.exp(s16)
    ex128 = jnp.dot(ex16, expand[...], preferred_element_type=jnp.float32)
    exr[...] = ex128
    numr[...] = ex128 * v


def _edge(qd, kvs, ea, wekv, hsel, expand):
    r = 512
    grid = (E_PAD // r,)
    return pl.pallas_call(
        _edge_body,
        grid=grid,
        in_specs=[
            pl.BlockSpec((r, HID), lambda i: (i, 0)),
            pl.BlockSpec((r, 256), lambda i: (i, 0)),
            pl.BlockSpec((r, ED), lambda i: (i, 0)),
            pl.BlockSpec((ED, 256), lambda i: (0, 0)),
            pl.BlockSpec((HID, 16), lambda i: (0, 0)),
            pl.BlockSpec((16, HID), lambda i: (0, 0)),
        ],
        out_specs=[
            pl.BlockSpec((r, HID), lambda i: (i, 0)),
            pl.BlockSpec((r, HID), lambda i: (i, 0)),
        ],
        out_shape=[
            jax.ShapeDtypeStruct((E_PAD, HID), jnp.float32),
            jax.ShapeDtypeStruct((E_PAD, HID), jnp.float32),
        ],
    )(qd, kvs, ea, wekv, hsel, expand)


def _node_body(x, n0, n1, n2, d0, d1, d2, wo, ln1s, ln1b, w1, b1, w2,
               b2, ln2s, ln2b, out):
    agg = (n0[...] / (d0[...] + 1e-9)
           + n1[...] / (d1[...] + 1e-9)
           + n2[...] / (d2[...] + 1e-9))
    h1 = x[...] + jnp.dot(agg, wo[...], preferred_element_type=jnp.float32)
    mu = jnp.mean(h1, axis=1, keepdims=True)
    var = jnp.mean((h1 - mu) ** 2, axis=1, keepdims=True)
    h = (h1 - mu) * lax.rsqrt(var + 1e-5) * ln1s[...] + ln1b[...]
    f = jnp.maximum(jnp.dot(h, w1[...], preferred_element_type=jnp.float32) + b1[...], 0.0)
    f = jnp.dot(f, w2[...], preferred_element_type=jnp.float32) + b2[...]
    h2 = h + f
    mu2 = jnp.mean(h2, axis=1, keepdims=True)
    var2 = jnp.mean((h2 - mu2) ** 2, axis=1, keepdims=True)
    out[...] = (h2 - mu2) * lax.rsqrt(var2 + 1e-5) * ln2s[...] + ln2b[...]


def _node(x, n0, n1, n2, d0, d1, d2, npar):
    r = 200
    grid = (N // r,)
    row = lambda i: (i, 0)
    fix = lambda i: (0, 0)
    return pl.pallas_call(
        _node_body,
        grid=grid,
        in_specs=[
            pl.BlockSpec((r, HID), row),
            pl.BlockSpec((r, HID), row),
            pl.BlockSpec((r, HID), row),
            pl.BlockSpec((r, HID), row),
            pl.BlockSpec((r, HID), row),
            pl.BlockSpec((r, HID), row),
            pl.BlockSpec((r, HID), row),
            pl.BlockSpec((HID, HID), fix),
            pl.BlockSpec((1, HID), fix),
            pl.BlockSpec((1, HID), fix),
            pl.BlockSpec((HID, 4 * HID), fix),
            pl.BlockSpec((1, 4 * HID), fix),
            pl.BlockSpec((4 * HID, HID), fix),
            pl.BlockSpec((1, HID), fix),
            pl.BlockSpec((1, HID), fix),
            pl.BlockSpec((1, HID), fix),
        ],
        out_specs=pl.BlockSpec((r, HID), row),
        out_shape=jax.ShapeDtypeStruct((N, HID), jnp.float32),
    )(
        x, n0, n1, n2, d0, d1, d2, npar["Wo"],
        npar["ln1_s"].reshape(1, HID), npar["ln1_b"].reshape(1, HID),
        npar["W1"], npar["b1"].reshape(1, 4 * HID),
        npar["W2"], npar["b2"].reshape(1, HID),
        npar["ln2_s"].reshape(1, HID), npar["ln2_b"].reshape(1, HID),
    )


@functools.lru_cache(maxsize=None)
def _sc_gather_kernel():
    from jax.experimental.pallas import tpu_sc as plsc

    mesh = plsc.VectorSubcoreMesh(core_axis_name="c", subcore_axis_name="s")

    @functools.partial(
        pl.kernel,
        out_type=(
            jax.ShapeDtypeStruct((E_PAD, HID), jnp.float32),
            jax.ShapeDtypeStruct((E_PAD, 256), jnp.float32),
        ),
        mesh=mesh,
        scratch_types=[
            pltpu.VMEM((G_STEPS, CH), jnp.int32),
            pltpu.VMEM((G_STEPS, CH), jnp.int32),
            pltpu.VMEM((CH, HID), jnp.float32),
            pltpu.VMEM((CH, 256), jnp.float32),
            pltpu.SemaphoreType.DMA,
            pltpu.SemaphoreType.DMA,
        ],
    )
    def gather(qtab, kvtab, dst2, src2, qd_out, kvs_out, idxd, idxs, qbuf,
               kvbuf, sq, skv):
        wid = lax.axis_index("s") * 2 + lax.axis_index("c")
        base = wid * G_PER_W
        pltpu.sync_copy(dst2.at[wid], idxd)
        pltpu.sync_copy(src2.at[wid], idxs)

        def body(ci, carry):
            off = base + ci * CH
            cq = pltpu.async_copy(qtab.at[idxd.at[ci]], qbuf, sq)
            ck = pltpu.async_copy(kvtab.at[idxs.at[ci]], kvbuf, skv)
            cq.wait()
            ck.wait()
            pltpu.sync_copy(qbuf, qd_out.at[pl.ds(off, CH)])
            pltpu.sync_copy(kvbuf, kvs_out.at[pl.ds(off, CH)])
            return carry

        lax.fori_loop(0, G_STEPS, body, 0)

    return gather


@functools.lru_cache(maxsize=None)
def _sc_scatter_kernel(width):
    from jax.experimental.pallas import tpu_sc as plsc

    mesh = plsc.VectorSubcoreMesh(core_axis_name="c", subcore_axis_name="s")

    @functools.partial(
        pl.kernel,
        out_type=jax.ShapeDtypeStruct((N_PAD, width), jnp.float32),
        mesh=mesh,
        scratch_types=[
            pltpu.VMEM((1, CH), jnp.int32),
            pltpu.VMEM((CH, width), jnp.float32),
            pltpu.VMEM_SHARED((ACC, width), jnp.float32),
        ],
    )
    def scatter(rows, idx2, zrow, out, idxl, rbuf, accn):
        c = lax.axis_index("c")
        s = lax.axis_index("s")
        nbase = c * NHALF
        pltpu.sync_copy(zrow.at[pl.ds(0, TPS)], accn.at[pl.ds(s * TPS, TPS)])

        @pl.when(s == 0)
        def _zero_dump():
            pltpu.sync_copy(zrow.at[pl.ds(TPS, 8)], accn.at[pl.ds(NHALF, 8)])

        plsc.subcore_barrier()

        def body(ci, carry):
            off = s * S_PER_T + ci * CH
            pltpu.sync_copy(idx2.at[c, pl.ds(off, CH)], idxl.at[0])
            pltpu.sync_copy(rows.at[pl.ds(off, CH)], rbuf)
            pltpu.sync_copy(rbuf, accn.at[idxl.at[0]], add=True)
            return carry

        lax.fori_loop(0, S_STEPS, body, 0)
        plsc.subcore_barrier()
        pltpu.sync_copy(accn.at[pl.ds(s * TPS, TPS)],
                        out.at[pl.ds(nbase + s * TPS, TPS)])

    return scatter


def kernel(all_gripper_feats_left, all_gripper_feats_right, edge_index,
           edge_attr, params, current_start_left, current_end_left,
           current_start_right, current_end_right):
    hsel = jnp.asarray(_np_hsel())
    expand = jnp.asarray(_np_expand())
    zn = jnp.zeros((TPS + 8, HID), jnp.float32)

    pad = E_PAD - E
    srcs, dst_g, dst_s, ea_p = {}, {}, {}, {}
    for name, _, _ in TYPE_TRIPLES:
        srcs[name] = jnp.pad(edge_index[name][0],
                             (0, pad)).reshape(32, G_STEPS, CH)
        dst_g[name] = jnp.pad(edge_index[name][1],
                              (0, pad)).reshape(32, G_STEPS, CH)
        dstp = jnp.pad(edge_index[name][1], (0, pad),
                       constant_values=SENTINEL)
        halves = []
        for cidx in (0, 1):
            lo = cidx * NHALF
            loc = dstp - lo
            ok = (dstp >= lo) & (dstp < lo + NHALF)
            halves.append(jnp.where(ok, loc, DUMP))
        dst_s[name] = jnp.stack(halves).astype(jnp.int32)
        ea_p[name] = jnp.pad(edge_attr[name], ((0, pad), (0, 0)))

    gather = _sc_gather_kernel()
    scatter_add = _sc_scatter_kernel(HID)

    x = {"left": all_gripper_feats_left, "right": all_gripper_feats_right}
    for lp in params["layers"]:
        tabs = {}
        for side in ("left", "right"):
            dst_ts = [t for t, _, d in TYPE_TRIPLES if d == side]
            src_ts = [t for t, s, _ in TYPE_TRIPLES if s == side]
            wcat = jnp.concatenate(
                [lp[t]["Wq"] for t in dst_ts]
                + [w for t in src_ts for w in (lp[t]["Wk"], lp[t]["Wv"])],
                axis=1)
            outs = _proj(x[side], wcat)
            for t, qt in zip(dst_ts, outs[:3]):
                tabs[("q", t)] = qt
            for t, kvt in zip(src_ts, outs[3:]):
                tabs[("kv", t)] = kvt

        aggs = {"left": [], "right": []}
        for name, st, dt in TYPE_TRIPLES:
            qd, kvs = gather(tabs[("q", name)], tabs[("kv", name)],
                             dst_g[name], srcs[name])
            wekv = jnp.concatenate([lp[name]["We_k"], lp[name]["We_v"]],
                                   axis=1)
            numr, exr = _edge(qd, kvs, ea_p[name], wekv, hsel, expand)
            num = scatter_add(numr, dst_s[name], zn)
            den = scatter_add(exr, dst_s[name], zn)
            aggs[dt].append((num[:N], den[:N]))

        newx = {}
        for side in ("left", "right"):
            (n0, d0), (n1, d1), (n2, d2) = aggs[side]
            newx[side] = _node(x[side], n0, n1, n2, d0, d1, d2,
                               lp["node_" + side])
        x = newx

    bl = lax.dynamic_slice_in_dim(x["left"], current_end_left - 1000, 1000, axis=0)
    br = lax.dynamic_slice_in_dim(x["right"], current_end_right - 1000, 1000, axis=0)
    return (bl, br)


# speedup vs baseline: 1.2996x; 1.0756x over previous
import functools

import jax
import jax.numpy as jnp
import numpy as np
from jax import lax
from jax.experimental import pallas as pl
from jax.experimental.pallas import tpu as pltpu

HID = 128
HEADS = 4
HD = 32
ED = 16
N = 25000
E = 100000
E_PAD = 102400
CH = 128
G_PER_W = 3200
G_STEPS = G_PER_W // CH
S_PER_T = E_PAD // 16
S_STEPS = S_PER_T // CH
NHALF = 12800
DUMP = NHALF
ACC = NHALF + 8
N_PAD = 2 * NHALF
TPS = NHALF // 16
SENTINEL = 1 << 28

TYPE_TRIPLES = [
    ("ll_temporal", "left", "left"),
    ("rr_temporal", "right", "right"),
    ("ll_context", "left", "left"),
    ("rr_context", "right", "right"),
    ("lr_bimanual", "left", "right"),
    ("rl_bimanual", "right", "left"),
]


def _np_hsel():
    m = np.zeros((HID, 16), np.float32)
    for d in range(HID):
        m[d, d // HD] = 1.0 / np.sqrt(HD)
    return m


def _np_expand():
    m = np.zeros((16, HID), np.float32)
    for d in range(HID):
        m[d // HD, d] = 1.0
    return m


def _proj_body(x, w, q0, q1, q2, kv0, kv1, kv2):
    y = jnp.dot(x[...], w[...], preferred_element_type=jnp.float32)
    q0[...] = y[:, 0:128]
    q1[...] = y[:, 128:256]
    q2[...] = y[:, 256:384]
    kv0[...] = y[:, 384:640]
    kv1[...] = y[:, 640:896]
    kv2[...] = y[:, 896:1152]


def _proj(x, wcat):
    r = 200
    grid = (N // r,)
    return pl.pallas_call(
        _proj_body,
        grid=grid,
        in_specs=[
            pl.BlockSpec((r, HID), lambda i: (i, 0)),
            pl.BlockSpec((HID, 1152), lambda i: (0, 0)),
        ],
        out_specs=[pl.BlockSpec((r, HID), lambda i: (i, 0))] * 3
        + [pl.BlockSpec((r, 256), lambda i: (i, 0))] * 3,
        out_shape=[jax.ShapeDtypeStruct((N, HID), jnp.float32)] * 3
        + [jax.ShapeDtypeStruct((N, 256), jnp.float32)] * 3,
    )(x, wcat)


def _edge_body(qd, kvs, ea, wekv, hsel, expand, numr, exr):
    kv = kvs[...] + jnp.dot(ea[...], wekv[...], preferred_element_type=jnp.float32)
    k = kv[:, :HID]
    v = kv[:, HID:]
    s16 = jnp.dot(qd[...] * k, hsel[...], preferred_element_type=jnp.float32)
    ex16 = jnp.exp(s16)
    ex128 = jnp.dot(ex16, expand[...], preferred_element_type=jnp.float32)
    exr[...] = ex128
    numr[...] = ex128 * v


def _edge(qd, kvs, ea, wekv, hsel, expand):
    r = 512
    grid = (E_PAD // r,)
    return pl.pallas_call(
        _edge_body,
        grid=grid,
        in_specs=[
            pl.BlockSpec((r, HID), lambda i: (i, 0)),
            pl.BlockSpec((r, 256), lambda i: (i, 0)),
            pl.BlockSpec((r, ED), lambda i: (i, 0)),
            pl.BlockSpec((ED, 256), lambda i: (0, 0)),
            pl.BlockSpec((HID, 16), lambda i: (0, 0)),
            pl.BlockSpec((16, HID), lambda i: (0, 0)),
        ],
        out_specs=[
            pl.BlockSpec((r, HID), lambda i: (i, 0)),
            pl.BlockSpec((r, HID), lambda i: (i, 0)),
        ],
        out_shape=[
            jax.ShapeDtypeStruct((E_PAD, HID), jnp.float32),
            jax.ShapeDtypeStruct((E_PAD, HID), jnp.float32),
        ],
    )(qd, kvs, ea, wekv, hsel, expand)


def _node_body(x, n0, n1, n2, d0, d1, d2, wo, ln1s, ln1b, w1, b1, w2,
               b2, ln2s, ln2b, out):
    agg = (n0[...] / (d0[...] + 1e-9)
           + n1[...] / (d1[...] + 1e-9)
           + n2[...] / (d2[...] + 1e-9))
    h1 = x[...] + jnp.dot(agg, wo[...], preferred_element_type=jnp.float32)
    mu = jnp.mean(h1, axis=1, keepdims=True)
    var = jnp.mean((h1 - mu) ** 2, axis=1, keepdims=True)
    h = (h1 - mu) * lax.rsqrt(var + 1e-5) * ln1s[...] + ln1b[...]
    f = jnp.maximum(jnp.dot(h, w1[...], preferred_element_type=jnp.float32) + b1[...], 0.0)
    f = jnp.dot(f, w2[...], preferred_element_type=jnp.float32) + b2[...]
    h2 = h + f
    mu2 = jnp.mean(h2, axis=1, keepdims=True)
    var2 = jnp.mean((h2 - mu2) ** 2, axis=1, keepdims=True)
    out[...] = (h2 - mu2) * lax.rsqrt(var2 + 1e-5) * ln2s[...] + ln2b[...]


def _node(x, n0, n1, n2, d0, d1, d2, npar):
    r = 200
    grid = (N // r,)
    row = lambda i: (i, 0)
    fix = lambda i: (0, 0)
    return pl.pallas_call(
        _node_body,
        grid=grid,
        in_specs=[
            pl.BlockSpec((r, HID), row),
            pl.BlockSpec((r, HID), row),
            pl.BlockSpec((r, HID), row),
            pl.BlockSpec((r, HID), row),
            pl.BlockSpec((r, HID), row),
            pl.BlockSpec((r, HID), row),
            pl.BlockSpec((r, HID), row),
            pl.BlockSpec((HID, HID), fix),
            pl.BlockSpec((1, HID), fix),
            pl.BlockSpec((1, HID), fix),
            pl.BlockSpec((HID, 4 * HID), fix),
            pl.BlockSpec((1, 4 * HID), fix),
            pl.BlockSpec((4 * HID, HID), fix),
            pl.BlockSpec((1, HID), fix),
            pl.BlockSpec((1, HID), fix),
            pl.BlockSpec((1, HID), fix),
        ],
        out_specs=pl.BlockSpec((r, HID), row),
        out_shape=jax.ShapeDtypeStruct((N, HID), jnp.float32),
    )(
        x, n0, n1, n2, d0, d1, d2, npar["Wo"],
        npar["ln1_s"].reshape(1, HID), npar["ln1_b"].reshape(1, HID),
        npar["W1"], npar["b1"].reshape(1, 4 * HID),
        npar["W2"], npar["b2"].reshape(1, HID),
        npar["ln2_s"].reshape(1, HID), npar["ln2_b"].reshape(1, HID),
    )


@functools.lru_cache(maxsize=None)
def _sc_gather_kernel():
    from jax.experimental.pallas import tpu_sc as plsc

    mesh = plsc.VectorSubcoreMesh(core_axis_name="c", subcore_axis_name="s")

    @functools.partial(
        pl.kernel,
        out_type=(
            jax.ShapeDtypeStruct((E_PAD, HID), jnp.float32),
            jax.ShapeDtypeStruct((E_PAD, 256), jnp.float32),
        ),
        mesh=mesh,
        scratch_types=[
            pltpu.VMEM((G_STEPS, CH), jnp.int32),
            pltpu.VMEM((G_STEPS, CH), jnp.int32),
            pltpu.VMEM((CH, HID), jnp.float32),
            pltpu.VMEM((CH, 256), jnp.float32),
            pltpu.SemaphoreType.DMA,
            pltpu.SemaphoreType.DMA,
        ],
    )
    def gather(qtab, kvtab, dst2, src2, qd_out, kvs_out, idxd, idxs, qbuf,
               kvbuf, sq, skv):
        wid = lax.axis_index("s") * 2 + lax.axis_index("c")
        base = wid * G_PER_W
        pltpu.sync_copy(dst2.at[wid], idxd)
        pltpu.sync_copy(src2.at[wid], idxs)

        def body(ci, carry):
            off = base + ci * CH
            cq = pltpu.async_copy(qtab.at[idxd.at[ci]], qbuf, sq)
            ck = pltpu.async_copy(kvtab.at[idxs.at[ci]], kvbuf, skv)
            cq.wait()
            ck.wait()
            pltpu.sync_copy(qbuf, qd_out.at[pl.ds(off, CH)])
            pltpu.sync_copy(kvbuf, kvs_out.at[pl.ds(off, CH)])
            return carry

        lax.fori_loop(0, G_STEPS, body, 0)

    return gather


@functools.lru_cache(maxsize=None)
def _sc_scatter_kernel(width):
    from jax.experimental.pallas import tpu_sc as plsc

    mesh = plsc.VectorSubcoreMesh(core_axis_name="c", subcore_axis_name="s")

    @functools.partial(
        pl.kernel,
        out_type=jax.ShapeDtypeStruct((N_PAD, width), jnp.float32),
        mesh=mesh,
        scratch_types=[
            pltpu.VMEM((S_STEPS, CH), jnp.int32),
            pltpu.VMEM((CH, width), jnp.float32),
            pltpu.VMEM_SHARED((ACC, width), jnp.float32),
        ],
    )
    def scatter(rows, idx2, zrow, out, idxl, rbuf, accn):
        c = lax.axis_index("c")
        s = lax.axis_index("s")
        nbase = c * NHALF
        pltpu.sync_copy(idx2.at[c, s], idxl)
        pltpu.sync_copy(zrow.at[pl.ds(0, TPS)], accn.at[pl.ds(s * TPS, TPS)])

        @pl.when(s == 0)
        def _zero_dump():
            pltpu.sync_copy(zrow.at[pl.ds(TPS, 8)], accn.at[pl.ds(NHALF, 8)])

        plsc.subcore_barrier()

        def body(ci, carry):
            off = s * S_PER_T + ci * CH
            pltpu.sync_copy(rows.at[pl.ds(off, CH)], rbuf)
            pltpu.sync_copy(rbuf, accn.at[idxl.at[ci]], add=True)
            return carry

        lax.fori_loop(0, S_STEPS, body, 0)
        plsc.subcore_barrier()
        pltpu.sync_copy(accn.at[pl.ds(s * TPS, TPS)],
                        out.at[pl.ds(nbase + s * TPS, TPS)])

    return scatter


def kernel(all_gripper_feats_left, all_gripper_feats_right, edge_index,
           edge_attr, params, current_start_left, current_end_left,
           current_start_right, current_end_right):
    hsel = jnp.asarray(_np_hsel())
    expand = jnp.asarray(_np_expand())
    zn = jnp.zeros((TPS + 8, HID), jnp.float32)

    pad = E_PAD - E
    srcs, dst_g, dst_s, ea_p = {}, {}, {}, {}
    for name, _, _ in TYPE_TRIPLES:
        srcs[name] = jnp.pad(edge_index[name][0],
                             (0, pad)).reshape(32, G_STEPS, CH)
        dst_g[name] = jnp.pad(edge_index[name][1],
                              (0, pad)).reshape(32, G_STEPS, CH)
        dstp = jnp.pad(edge_index[name][1], (0, pad),
                       constant_values=SENTINEL)
        halves = []
        for cidx in (0, 1):
            lo = cidx * NHALF
            loc = dstp - lo
            ok = (dstp >= lo) & (dstp < lo + NHALF)
            halves.append(jnp.where(ok, loc, DUMP))
        dst_s[name] = jnp.stack(halves).astype(jnp.int32).reshape(
            2, 16, S_STEPS, CH)
        ea_p[name] = jnp.pad(edge_attr[name], ((0, pad), (0, 0)))

    gather = _sc_gather_kernel()
    scatter_add = _sc_scatter_kernel(HID)

    x = {"left": all_gripper_feats_left, "right": all_gripper_feats_right}
    for lp in params["layers"]:
        tabs = {}
        for side in ("left", "right"):
            dst_ts = [t for t, _, d in TYPE_TRIPLES if d == side]
            src_ts = [t for t, s, _ in TYPE_TRIPLES if s == side]
            wcat = jnp.concatenate(
                [lp[t]["Wq"] for t in dst_ts]
                + [w for t in src_ts for w in (lp[t]["Wk"], lp[t]["Wv"])],
                axis=1)
            outs = _proj(x[side], wcat)
            for t, qt in zip(dst_ts, outs[:3]):
                tabs[("q", t)] = qt
            for t, kvt in zip(src_ts, outs[3:]):
                tabs[("kv", t)] = kvt

        aggs = {"left": [], "right": []}
        for name, st, dt in TYPE_TRIPLES:
            qd, kvs = gather(tabs[("q", name)], tabs[("kv", name)],
                             dst_g[name], srcs[name])
            wekv = jnp.concatenate([lp[name]["We_k"], lp[name]["We_v"]],
                                   axis=1)
            numr, exr = _edge(qd, kvs, ea_p[name], wekv, hsel, expand)
            num = scatter_add(numr, dst_s[name], zn)
            den = scatter_add(exr, dst_s[name], zn)
            aggs[dt].append((num[:N], den[:N]))

        newx = {}
        for side in ("left", "right"):
            (n0, d0), (n1, d1), (n2, d2) = aggs[side]
            newx[side] = _node(x[side], n0, n1, n2, d0, d1, d2,
                               lp["node_" + side])
        x = newx

    bl = lax.dynamic_slice_in_dim(x["left"], current_end_left - 1000, 1000, axis=0)
    br = lax.dynamic_slice_in_dim(x["right"], current_end_right - 1000, 1000, axis=0)
    return (bl, br)
